# Initial kernel scaffold; baseline (speedup 1.0000x reference)
import functools

import jax
import jax.numpy as jnp
from jax import lax
from jax.experimental import pallas as pl
from jax.experimental.pallas import tpu as pltpu
from jax.experimental.pallas import tpu_sc as plsc

NBOX = 1000
L = 16
NPAD = 1008
NCHUNK = NPAD // L
NB = 4
NC = 21
NCLS = NC - 1
NPAIR = NB * NCLS
NWORK = 32
TH_CONF = 0.05
TH_IOU = 0.5
NEG = float("-inf")
BIGI = jnp.int32(2**30)


def _body(conf_hbm, loc_hbm, anch_hbm, out_hbm,
          conf_v, cls_v, loc_v, anch_v, comp_v, cidx_v, out_v):
    cid = lax.axis_index("c")
    sid = lax.axis_index("s")
    wid = sid * 2 + cid
    lane = lax.iota(jnp.int32, (L,))
    lane0 = lane == 0

    pltpu.sync_copy(anch_hbm, anch_v)

    def do_pair(pair):
        b = pair // NCLS
        cls = pair % NCLS + 1
        pltpu.sync_copy(conf_hbm.at[b], conf_v)
        pltpu.sync_copy(conf_hbm.at[b, cls], cls_v)
        pltpu.sync_copy(loc_hbm.at[b], loc_v)

        def chunk_body(i, cnt):
            sl = pl.ds(i * L, L)
            m = conf_v[0, sl]
            for c in range(1, NC):
                m = jnp.maximum(m, conf_v[c, sl])
            z = jnp.exp(conf_v[0, sl] - m)
            for c in range(1, NC):
                z = z + jnp.exp(conf_v[c, sl] - m)
            s = jnp.exp(cls_v[sl] - m) / z

            a0 = anch_v[0, sl]
            a1 = anch_v[1, sl]
            a2 = anch_v[2, sl]
            a3 = anch_v[3, sl]
            cx = a0 + loc_v[0, sl] * 0.1 * a2
            cy = a1 + loc_v[1, sl] * 0.1 * a3
            w = a2 * jnp.exp(loc_v[2, sl] * 0.2)
            h = a3 * jnp.exp(loc_v[3, sl] * 0.2)
            x1 = cx - w / 2.0
            y1 = cy - h / 2.0
            x2 = cx + w / 2.0
            y2 = cy + h / 2.0
            area = (x2 - x1) * (y2 - y1)

            mask = s >= TH_CONF
            off = pl.ds(cnt, L)
            plsc.store_compressed(comp_v.at[0, off], x1, mask=mask)
            plsc.store_compressed(comp_v.at[1, off], y1, mask=mask)
            plsc.store_compressed(comp_v.at[2, off], x2, mask=mask)
            plsc.store_compressed(comp_v.at[3, off], y2, mask=mask)
            plsc.store_compressed(comp_v.at[4, off], area, mask=mask)
            plsc.store_compressed(comp_v.at[5, off], s, mask=mask)
            plsc.store_compressed(comp_v.at[6, off], s, mask=mask)
            plsc.store_compressed(cidx_v.at[off], lane + i * L, mask=mask)

            zv = jnp.zeros((L,), jnp.float32)
            for j in range(5):
                out_v[j, sl] = zv

            pc = plsc.all_reduce_population_count(mask)
            return cnt + jnp.max(pc)

        cnt = lax.fori_loop(0, NCHUNK, chunk_body, jnp.int32(0))

        off = pl.ds(cnt, L)
        comp_v[5, off] = jnp.full((L,), NEG, jnp.float32)
        comp_v[6, off] = jnp.full((L,), NEG, jnp.float32)
        cidx_v[off] = jnp.full((L,), BIGI, jnp.int32)

        nv = (cnt + (L - 1)) // L

        def find_pos(mval):
            def pb(v, carry):
                bi, bp = carry
                sl2 = pl.ds(v * L, L)
                sv = comp_v[5, sl2]
                iv = cidx_v[sl2]
                cand = jnp.where(sv == mval, iv, BIGI)
                mi = jnp.min(cand)
                p2 = jnp.min(jnp.where(cand == mi, lane + v * L, BIGI))
                better = mi < bi
                return (jnp.where(better, mi, bi), jnp.where(better, p2, bp))
            return lax.fori_loop(0, nv, pb, (BIGI, jnp.int32(0)))

        def find_max():
            def mb(v, acc):
                return jnp.maximum(acc, comp_v[5, pl.ds(v * L, L)])
            mvec = lax.fori_loop(0, nv, mb, jnp.full((L,), NEG, jnp.float32))
            return jnp.max(mvec)

        mval0 = find_max()
        bidx0, bpos0 = find_pos(mval0)

        def cond(st):
            return st[0] > jnp.float32(-1e38)

        def sel_body(st):
            mval, bidx, bpos = st
            pv = jnp.full((L,), bpos, jnp.int32)
            x1s = plsc.load_gather(comp_v.at[0], [pv])
            y1s = plsc.load_gather(comp_v.at[1], [pv])
            x2s = plsc.load_gather(comp_v.at[2], [pv])
            y2s = plsc.load_gather(comp_v.at[3], [pv])
            ars = plsc.load_gather(comp_v.at[4], [pv])

            def rb(v, r):
                sl2 = pl.ds(v * L, L)
                s0 = comp_v[6, sl2]
                iv = cidx_v[sl2]
                before = (s0 > mval) | ((s0 == mval) & (iv < bidx))
                return r + jnp.max(plsc.all_reduce_population_count(before))
            rank = lax.fori_loop(0, nv, rb, jnp.int32(0))

            rv = jnp.full((L,), rank, jnp.int32)
            plsc.store_scatter(out_v.at[0], [rv], x1s, mask=lane0)
            plsc.store_scatter(out_v.at[1], [rv], y1s, mask=lane0)
            plsc.store_scatter(out_v.at[2], [rv], x2s, mask=lane0)
            plsc.store_scatter(out_v.at[3], [rv], y2s, mask=lane0)
            plsc.store_scatter(out_v.at[4], [rv],
                               jnp.full((L,), mval, jnp.float32), mask=lane0)

            def sb(v, acc):
                sl2 = pl.ds(v * L, L)
                x1 = comp_v[0, sl2]
                y1 = comp_v[1, sl2]
                x2 = comp_v[2, sl2]
                y2 = comp_v[3, sl2]
                ar = comp_v[4, sl2]
                ix1 = jnp.maximum(x1s, x1)
                iy1 = jnp.maximum(y1s, y1)
                ix2 = jnp.minimum(x2s, x2)
                iy2 = jnp.minimum(y2s, y2)
                inter = jnp.maximum(ix2 - ix1, 0.0) * jnp.maximum(iy2 - iy1, 0.0)
                iou = inter / jnp.maximum(ars + ar - inter, 1e-9)
                sv = comp_v[5, sl2]
                sv = jnp.where(iou > TH_IOU, NEG, sv)
                comp_v[5, sl2] = sv
                return jnp.maximum(acc, sv)
            mvec = lax.fori_loop(0, nv, sb, jnp.full((L,), NEG, jnp.float32))
            nmval = jnp.max(mvec)
            nbidx, nbpos = find_pos(nmval)
            return (nmval, nbidx, nbpos)

        lax.while_loop(cond, sel_body, (mval0, bidx0, bpos0))

        pltpu.sync_copy(out_v, out_hbm.at[pair])

    def pair_loop(t, _):
        pair = wid + t * NWORK

        @pl.when(pair < NPAIR)
        def _():
            do_pair(pair)
        return jnp.int32(0)

    lax.fori_loop(0, (NPAIR + NWORK - 1) // NWORK, pair_loop, jnp.int32(0))


@jax.jit
def kernel(conf, loc, anchors):
    padn = NPAD - NBOX
    pad_cls = jnp.where(lax.iota(jnp.float32, (NC,)) == 0, 100.0, -100.0)
    conf_p = jnp.concatenate(
        [conf, jnp.broadcast_to(pad_cls, (NB, padn, NC))], axis=1)
    conf_t = jnp.transpose(conf_p, (0, 2, 1))
    loc_t = jnp.transpose(
        jnp.pad(loc, ((0, 0), (0, padn), (0, 0))), (0, 2, 1))
    anch_t = jnp.transpose(
        jnp.pad(anchors, ((0, padn), (0, 0))), (1, 0))

    mesh = plsc.VectorSubcoreMesh(core_axis_name="c", subcore_axis_name="s",
                                  num_cores=2, num_subcores=16)
    out = pl.kernel(
        _body,
        out_type=jax.ShapeDtypeStruct((NPAIR, 5, NPAD), jnp.float32),
        mesh=mesh,
        scratch_types=[
            pltpu.VMEM((NC, NPAD), jnp.float32),
            pltpu.VMEM((NPAD,), jnp.float32),
            pltpu.VMEM((4, NPAD), jnp.float32),
            pltpu.VMEM((4, NPAD), jnp.float32),
            pltpu.VMEM((7, NPAD + L), jnp.float32),
            pltpu.VMEM((NPAD + L,), jnp.int32),
            pltpu.VMEM((5, NPAD), jnp.float32),
        ],
    )(conf_t, loc_t, anch_t)

    return (out[:, :, :NBOX]
            .reshape(NB, NCLS, 5, NBOX)
            .transpose(0, 1, 3, 2))


# speedup vs baseline: 20.2292x; 20.2292x over previous
import functools

import numpy as np
import jax
import jax.numpy as jnp
from jax import lax
from jax.experimental import pallas as pl
from jax.experimental.pallas import tpu as pltpu
from jax.experimental.pallas import tpu_sc as plsc

NBOX = 1000
L = 16
NPAD = 1024
NCHUNK = NPAD // L
NB = 4
NC = 21
NCLS = NC - 1
NPAIR = NB * NCLS
NWORK = 32
TH_CONF = 0.05
TH_IOU = 0.5
NEG = float("-inf")
BIGI = np.int32(2**30)


def _body(conf_hbm, loc_hbm, anch_hbm, out_hbm,
          conf_v, cls_v, loc_v, anch_v,
          cx1_v, cy1_v, cx2_v, cy2_v, car_v, cs_v, cs0_v, cidx_v,
          o0_v, o1_v, o2_v, o3_v, o4_v, acc_v):
    cid = lax.axis_index("c")
    sid = lax.axis_index("s")
    wid = sid * 2 + cid
    lane = lax.iota(jnp.int32, L)
    lane0 = lane == 0

    pltpu.sync_copy(anch_hbm, anch_v)

    def do_pair(pair):
        b = pair // NCLS
        cls = pair % NCLS + 1
        pltpu.sync_copy(conf_hbm.at[b], conf_v)
        pltpu.sync_copy(conf_hbm.at[b, pl.ds(cls, 1)], cls_v)
        pltpu.sync_copy(loc_hbm.at[b], loc_v)

        def chunk_body(i, cnt):
            sl = pl.ds(i * L, L)
            m = conf_v[0, sl]
            for c in range(1, NC):
                m = jnp.maximum(m, conf_v[c, sl])
            z = jnp.exp(conf_v[0, sl] - m)
            for c in range(1, NC):
                z = z + jnp.exp(conf_v[c, sl] - m)
            s = jnp.exp(cls_v[0, sl] - m) / z

            a0 = anch_v[0, sl]
            a1 = anch_v[1, sl]
            a2 = anch_v[2, sl]
            a3 = anch_v[3, sl]
            cx = a0 + loc_v[0, sl] * 0.1 * a2
            cy = a1 + loc_v[1, sl] * 0.1 * a3
            w = a2 * jnp.exp(loc_v[2, sl] * 0.2)
            h = a3 * jnp.exp(loc_v[3, sl] * 0.2)
            x1 = cx - w / 2.0
            y1 = cy - h / 2.0
            x2 = cx + w / 2.0
            y2 = cy + h / 2.0
            area = (x2 - x1) * (y2 - y1)

            mask = s >= TH_CONF
            mi32 = mask.astype(jnp.int32)
            csum = lax.cumsum(mi32)
            pos = jnp.where(mask, cnt + csum - 1, NPAD + L - 1)
            plsc.store_scatter(cx1_v, [pos], x1)
            plsc.store_scatter(cy1_v, [pos], y1)
            plsc.store_scatter(cx2_v, [pos], x2)
            plsc.store_scatter(cy2_v, [pos], y2)
            plsc.store_scatter(car_v, [pos], area)
            plsc.store_scatter(cs_v, [pos], s)
            plsc.store_scatter(cs0_v, [pos], s)
            plsc.store_scatter(cidx_v, [pos], lane + i * L)

            zv = jnp.zeros((L,), jnp.float32)
            for o in (o0_v, o1_v, o2_v, o3_v, o4_v):
                o[0, sl] = zv

            return cnt + jnp.max(csum)

        cnt = lax.fori_loop(0, NCHUNK, chunk_body, np.int32(0))

        off = pl.ds(cnt, L)
        cs_v[off] = jnp.full((L,), NEG, jnp.float32)
        cs0_v[off] = jnp.full((L,), NEG, jnp.float32)
        cidx_v[off] = jnp.full((L,), BIGI, jnp.int32)

        nv = (cnt + (L - 1)) // L

        def find_pos(mval):
            def pb(v, carry):
                bi, bp = carry
                sl2 = pl.ds(v * L, L)
                sv = cs_v[sl2]
                iv = cidx_v[sl2]
                cand = jnp.where(sv == mval, iv, BIGI)
                mi = jnp.min(cand)
                p2 = jnp.min(jnp.where(cand == mi, lane + v * L, BIGI))
                better = mi < bi
                return (jnp.where(better, mi, bi), jnp.where(better, p2, bp))
            return lax.fori_loop(0, nv, pb, (BIGI, np.int32(0)))

        def find_max():
            acc_v[...] = jnp.full((L,), NEG, jnp.float32)

            def mb(v, _):
                acc_v[...] = jnp.maximum(acc_v[...], cs_v[pl.ds(v * L, L)])
                return np.int32(0)
            lax.fori_loop(0, nv, mb, np.int32(0))
            return jnp.max(acc_v[...])

        mval0 = find_max()
        bidx0, bpos0 = find_pos(mval0)

        def cond(st):
            return st[0] > np.float32(-1e38)

        def sel_body(st):
            mval, bidx, bpos = st
            pv = jnp.full((L,), bpos, jnp.int32)
            x1s = plsc.load_gather(cx1_v, [pv])
            y1s = plsc.load_gather(cy1_v, [pv])
            x2s = plsc.load_gather(cx2_v, [pv])
            y2s = plsc.load_gather(cy2_v, [pv])
            ars = plsc.load_gather(car_v, [pv])

            def rb(v, r):
                sl2 = pl.ds(v * L, L)
                s0 = cs0_v[sl2]
                iv = cidx_v[sl2]
                before = (s0 > mval) | ((s0 == mval) & (iv < bidx))
                return r + jnp.sum(before.astype(jnp.int32))
            rank = lax.fori_loop(0, nv, rb, np.int32(0))

            rv = jnp.full((L,), rank, jnp.int32)
            zrow = jnp.zeros((L,), jnp.int32)
            plsc.store_scatter(o0_v, [zrow, rv], x1s)
            plsc.store_scatter(o1_v, [zrow, rv], y1s)
            plsc.store_scatter(o2_v, [zrow, rv], x2s)
            plsc.store_scatter(o3_v, [zrow, rv], y2s)
            plsc.store_scatter(o4_v, [zrow, rv],
                               jnp.full((L,), mval, jnp.float32))

            acc_v[...] = jnp.full((L,), NEG, jnp.float32)

            def sb(v, _):
                sl2 = pl.ds(v * L, L)
                x1 = cx1_v[sl2]
                y1 = cy1_v[sl2]
                x2 = cx2_v[sl2]
                y2 = cy2_v[sl2]
                ar = car_v[sl2]
                ix1 = jnp.maximum(x1s, x1)
                iy1 = jnp.maximum(y1s, y1)
                ix2 = jnp.minimum(x2s, x2)
                iy2 = jnp.minimum(y2s, y2)
                inter = jnp.maximum(ix2 - ix1, 0.0) * jnp.maximum(iy2 - iy1, 0.0)
                iou = inter / jnp.maximum(ars + ar - inter, 1e-9)
                sv = cs_v[sl2]
                sv = jnp.where(iou > TH_IOU, NEG, sv)
                cs_v[sl2] = sv
                acc_v[...] = jnp.maximum(acc_v[...], sv)
                return np.int32(0)
            lax.fori_loop(0, nv, sb, np.int32(0))
            nmval = jnp.max(acc_v[...])
            nbidx, nbpos = find_pos(nmval)
            return (nmval, nbidx, nbpos)

        lax.while_loop(cond, sel_body, (mval0, bidx0, bpos0))

        for j, o in enumerate((o0_v, o1_v, o2_v, o3_v, o4_v)):
            pltpu.sync_copy(o, out_hbm.at[pair, pl.ds(j, 1)])

    def pair_loop(t, _):
        pair = wid + t * NWORK

        @pl.when(pair < NPAIR)
        def _():
            do_pair(pair)
        return np.int32(0)

    lax.fori_loop(0, (NPAIR + NWORK - 1) // NWORK, pair_loop, np.int32(0))


@jax.jit
def kernel(conf, loc, anchors):
    padn = NPAD - NBOX
    pad_cls = jnp.where(jnp.arange(NC) == 0, 100.0, -100.0).astype(jnp.float32)
    conf_p = jnp.concatenate(
        [conf, jnp.broadcast_to(pad_cls, (NB, padn, NC))], axis=1)
    conf_t = jnp.transpose(conf_p, (0, 2, 1))
    loc_t = jnp.transpose(
        jnp.pad(loc, ((0, 0), (0, padn), (0, 0))), (0, 2, 1))
    anch_t = jnp.transpose(
        jnp.pad(anchors, ((0, padn), (0, 0))), (1, 0))

    mesh = plsc.VectorSubcoreMesh(core_axis_name="c", subcore_axis_name="s",
                                  num_cores=2, num_subcores=16)
    out = pl.kernel(
        _body,
        out_type=jax.ShapeDtypeStruct((NPAIR, 5, NPAD), jnp.float32),
        mesh=mesh,
        compiler_params=pltpu.CompilerParams(needs_layout_passes=False),
        scratch_types=[
            pltpu.VMEM((NC, NPAD), jnp.float32),
            pltpu.VMEM((1, NPAD), jnp.float32),
            pltpu.VMEM((4, NPAD), jnp.float32),
            pltpu.VMEM((4, NPAD), jnp.float32),
        ] + [pltpu.VMEM((NPAD + L,), jnp.float32)] * 7
          + [pltpu.VMEM((NPAD + L,), jnp.int32)]
          + [pltpu.VMEM((1, NPAD), jnp.float32)] * 5
          + [pltpu.VMEM((L,), jnp.float32)],
    )(conf_t, loc_t, anch_t)

    return (out[:, :, :NBOX]
            .reshape(NB, NCLS, 5, NBOX)
            .transpose(0, 1, 3, 2))


# speedup vs baseline: 47.3328x; 2.3398x over previous
import functools

import numpy as np
import jax
import jax.numpy as jnp
from jax import lax
from jax.experimental import pallas as pl
from jax.experimental.pallas import tpu as pltpu
from jax.experimental.pallas import tpu_sc as plsc

NBOX = 1000
L = 16
NPAD = 1024
NCHUNK = NPAD // L
NB = 4
NC = 21
NCLS = NC - 1
NPAIR = NB * NCLS
NWORK = 32
TH_CONF = 0.05
TH_IOU = 0.5
NEG = float("-inf")
BIGI = np.int32(2**30)


def _body(conf_hbm, loc_hbm, anch_hbm, out_hbm,
          conf_v, cls_v, loc_v, anch_v,
          cx1_v, cy1_v, cx2_v, cy2_v, car_v, cs_v, cs0_v, cidx_v,
          o0_v, o1_v, o2_v, o3_v, o4_v):
    cid = lax.axis_index("c")
    sid = lax.axis_index("s")
    wid = sid * 2 + cid
    lane = lax.iota(jnp.int32, L)
    lane0 = lane == 0

    pltpu.sync_copy(anch_hbm, anch_v)

    def do_pair(pair):
        b = pair // NCLS
        cls = pair % NCLS + 1
        pltpu.sync_copy(conf_hbm.at[b], conf_v)
        pltpu.sync_copy(conf_hbm.at[b, pl.ds(cls, 1)], cls_v)
        pltpu.sync_copy(loc_hbm.at[b], loc_v)

        def chunk_body(i, cnt):
            sl = pl.ds(i * L, L)
            m = conf_v[0, sl]
            for c in range(1, NC):
                m = jnp.maximum(m, conf_v[c, sl])
            z = jnp.exp(conf_v[0, sl] - m)
            for c in range(1, NC):
                z = z + jnp.exp(conf_v[c, sl] - m)
            s = jnp.exp(cls_v[0, sl] - m) / z

            a0 = anch_v[0, sl]
            a1 = anch_v[1, sl]
            a2 = anch_v[2, sl]
            a3 = anch_v[3, sl]
            cx = a0 + loc_v[0, sl] * 0.1 * a2
            cy = a1 + loc_v[1, sl] * 0.1 * a3
            w = a2 * jnp.exp(loc_v[2, sl] * 0.2)
            h = a3 * jnp.exp(loc_v[3, sl] * 0.2)
            x1 = cx - w / 2.0
            y1 = cy - h / 2.0
            x2 = cx + w / 2.0
            y2 = cy + h / 2.0
            area = (x2 - x1) * (y2 - y1)

            mask = s >= TH_CONF
            mi32 = mask.astype(jnp.int32)
            csum = lax.cumsum(mi32)
            pos = jnp.where(mask, cnt + csum - 1, NPAD + 2 * L - 1)
            plsc.store_scatter(cx1_v, [pos], x1)
            plsc.store_scatter(cy1_v, [pos], y1)
            plsc.store_scatter(cx2_v, [pos], x2)
            plsc.store_scatter(cy2_v, [pos], y2)
            plsc.store_scatter(car_v, [pos], area)
            plsc.store_scatter(cs_v, [pos], s)
            plsc.store_scatter(cs0_v, [pos], s)
            plsc.store_scatter(cidx_v, [pos], lane + i * L)

            zv = jnp.zeros((L,), jnp.float32)
            for o in (o0_v, o1_v, o2_v, o3_v, o4_v):
                o[0, sl] = zv

            return cnt + jnp.max(csum)

        cnt = lax.fori_loop(0, NCHUNK, chunk_body, np.int32(0))

        negv = jnp.full((L,), NEG, jnp.float32)
        bigv = jnp.full((L,), BIGI, jnp.int32)
        for t in (0, L):
            off = pl.ds(cnt + t, L)
            cs_v[off] = negv
            cs0_v[off] = negv
            cidx_v[off] = bigv

        nh = (cnt + 2 * L - 1) // (2 * L)

        def arg_update(sv, iv, posv, best_s, best_i, best_p):
            c2 = (sv > best_s) | ((sv == best_s) & (iv < best_i))
            return (jnp.where(c2, sv, best_s),
                    jnp.where(c2, iv, best_i),
                    jnp.where(c2, posv, best_p))

        def finalize(best_s, best_i, best_p):
            mval = jnp.max(best_s)
            cand = jnp.where(best_s == mval, best_i, BIGI)
            bidx = jnp.min(cand)
            bpos = jnp.min(jnp.where(cand == bidx, best_p, BIGI))
            return (mval, bidx, bpos)

        zi = jnp.zeros((L,), jnp.int32)

        def ib(v, carry):
            best_s, best_i, best_p = carry
            for u in range(2):
                base = 2 * v * L + u * L
                sl2 = pl.ds(base, L)
                best_s, best_i, best_p = arg_update(
                    cs_v[sl2], cidx_v[sl2], lane + base, best_s, best_i, best_p)
            return (best_s, best_i, best_p)

        mval0, bidx0, bpos0 = finalize(
            *lax.fori_loop(0, nh, ib, (negv, bigv, zi)))

        def cond(st):
            return st[0] > np.float32(-1e38)

        def sel_body(st):
            mval, bidx, bpos = st
            pv = jnp.full((L,), bpos, jnp.int32)
            x1s = plsc.load_gather(cx1_v, [pv])
            y1s = plsc.load_gather(cy1_v, [pv])
            x2s = plsc.load_gather(cx2_v, [pv])
            y2s = plsc.load_gather(cy2_v, [pv])
            ars = plsc.load_gather(car_v, [pv])

            def fb(v, carry):
                best_s, best_i, best_p, rk = carry
                for u in range(2):
                    base = 2 * v * L + u * L
                    sl2 = pl.ds(base, L)
                    x1 = cx1_v[sl2]
                    y1 = cy1_v[sl2]
                    x2 = cx2_v[sl2]
                    y2 = cy2_v[sl2]
                    ar = car_v[sl2]
                    s0 = cs0_v[sl2]
                    iv = cidx_v[sl2]
                    sv = cs_v[sl2]
                    ix1 = jnp.maximum(x1s, x1)
                    iy1 = jnp.maximum(y1s, y1)
                    ix2 = jnp.minimum(x2s, x2)
                    iy2 = jnp.minimum(y2s, y2)
                    inter = (jnp.maximum(ix2 - ix1, 0.0)
                             * jnp.maximum(iy2 - iy1, 0.0))
                    union = jnp.maximum(ars + ar - inter, 1e-9)
                    sv = jnp.where(inter > TH_IOU * union, NEG, sv)
                    cs_v[sl2] = sv
                    before = (s0 > mval) | ((s0 == mval) & (iv < bidx))
                    rk = rk + jnp.where(before, 1, 0)
                    best_s, best_i, best_p = arg_update(
                        sv, iv, lane + base, best_s, best_i, best_p)
                return (best_s, best_i, best_p, rk)

            best_s, best_i, best_p, rk = lax.fori_loop(
                0, nh, fb, (negv, bigv, zi, zi))
            rank = jnp.sum(rk)

            rv = jnp.full((L,), rank, jnp.int32)
            zrow = jnp.zeros((L,), jnp.int32)
            plsc.store_scatter(o0_v, [zrow, rv], x1s)
            plsc.store_scatter(o1_v, [zrow, rv], y1s)
            plsc.store_scatter(o2_v, [zrow, rv], x2s)
            plsc.store_scatter(o3_v, [zrow, rv], y2s)
            plsc.store_scatter(o4_v, [zrow, rv],
                               jnp.full((L,), mval, jnp.float32))
            return finalize(best_s, best_i, best_p)

        lax.while_loop(cond, sel_body, (mval0, bidx0, bpos0))

        for j, o in enumerate((o0_v, o1_v, o2_v, o3_v, o4_v)):
            pltpu.sync_copy(o, out_hbm.at[pair, pl.ds(j, 1)])

    def pair_loop(t, _):
        pair = wid + t * NWORK

        @pl.when(pair < NPAIR)
        def _():
            do_pair(pair)
        return np.int32(0)

    lax.fori_loop(0, (NPAIR + NWORK - 1) // NWORK, pair_loop, np.int32(0))


@jax.jit
def kernel(conf, loc, anchors):
    padn = NPAD - NBOX
    pad_cls = jnp.where(jnp.arange(NC) == 0, 100.0, -100.0).astype(jnp.float32)
    conf_p = jnp.concatenate(
        [conf, jnp.broadcast_to(pad_cls, (NB, padn, NC))], axis=1)
    conf_t = jnp.transpose(conf_p, (0, 2, 1))
    loc_t = jnp.transpose(
        jnp.pad(loc, ((0, 0), (0, padn), (0, 0))), (0, 2, 1))
    anch_t = jnp.transpose(
        jnp.pad(anchors, ((0, padn), (0, 0))), (1, 0))

    mesh = plsc.VectorSubcoreMesh(core_axis_name="c", subcore_axis_name="s",
                                  num_cores=2, num_subcores=16)
    out = pl.kernel(
        _body,
        out_type=jax.ShapeDtypeStruct((NPAIR, 5, NPAD), jnp.float32),
        mesh=mesh,
        compiler_params=pltpu.CompilerParams(needs_layout_passes=False),
        scratch_types=[
            pltpu.VMEM((NC, NPAD), jnp.float32),
            pltpu.VMEM((1, NPAD), jnp.float32),
            pltpu.VMEM((4, NPAD), jnp.float32),
            pltpu.VMEM((4, NPAD), jnp.float32),
        ] + [pltpu.VMEM((NPAD + 2 * L,), jnp.float32)] * 7
          + [pltpu.VMEM((NPAD + 2 * L,), jnp.int32)]
          + [pltpu.VMEM((1, NPAD), jnp.float32)] * 5,
    )(conf_t, loc_t, anch_t)

    return (out[:, :, :NBOX]
            .reshape(NB, NCLS, 5, NBOX)
            .transpose(0, 1, 3, 2))
